# Initial kernel scaffold; baseline (speedup 1.0000x reference)
import jax
import jax.numpy as jnp
from jax.experimental import pallas as pl


def kernel(src, index, out):
    raise NotImplementedError("write your pallas kernel here")


# speedup vs baseline: 3.6755x; 3.6755x over previous
import functools

import jax
import jax.numpy as jnp
from jax import lax
from jax.experimental import pallas as pl
from jax.experimental.pallas import tpu as pltpu
from jax.experimental.pallas import tpu_sc as plsc

N_EDGES = 320000
N_NODES = 10000
D = 128

NC = 2
NS = 16
NW = NC * NS
N_PAD = 10112
EPW = N_EDGES // NW
CHUNK = 80
NCHUNKS = EPW // CHUNK
ROWS_PER_TILE = N_PAD // NS


def _sc_partial_kernel(src_hbm, idx_hbm, part_hbm, src_buf, idx_buf, acc_sh):
    c = lax.axis_index("c")
    s = lax.axis_index("s")
    wid = c * NS + s

    zeros16 = jnp.zeros((16,), jnp.float32)

    def zero_row(r, _):
        for k in range(D // 16):
            src_buf[r, pl.ds(k * 16, 16)] = zeros16
        return 0

    lax.fori_loop(0, CHUNK, zero_row, 0)

    row0 = s * ROWS_PER_TILE
    for j in range(ROWS_PER_TILE // CHUNK):
        pltpu.sync_copy(src_buf, acc_sh.at[pl.ds(row0 + j * CHUNK, CHUNK), :])
    rem = ROWS_PER_TILE % CHUNK
    pltpu.sync_copy(
        src_buf.at[pl.ds(0, rem), :],
        acc_sh.at[pl.ds(row0 + (ROWS_PER_TILE // CHUNK) * CHUNK, rem), :],
    )

    plsc.subcore_barrier()

    base = wid * EPW

    def body(t, _):
        e = base + t * CHUNK
        pltpu.sync_copy(idx_hbm.at[pl.ds(e, CHUNK)], idx_buf)
        pltpu.sync_copy(src_hbm.at[pl.ds(e, CHUNK), :], src_buf)
        pltpu.sync_copy(src_buf, acc_sh.at[idx_buf], add=True)
        return 0

    lax.fori_loop(0, NCHUNKS, body, 0)

    plsc.subcore_barrier()

    pltpu.sync_copy(
        acc_sh.at[pl.ds(row0, ROWS_PER_TILE), :],
        part_hbm.at[c, pl.ds(row0, ROWS_PER_TILE), :],
    )


@functools.partial(
    pl.kernel,
    mesh=plsc.VectorSubcoreMesh(core_axis_name="c", subcore_axis_name="s"),
    out_type=jax.ShapeDtypeStruct((NC, N_PAD, D), jnp.float32),
    scratch_types=[
        pltpu.VMEM((CHUNK, D), jnp.float32),
        pltpu.VMEM((CHUNK,), jnp.int32),
        pltpu.VMEM_SHARED((N_PAD, D), jnp.float32),
    ],
)
def _sc_partial(src_hbm, idx_hbm, part_hbm, src_buf, idx_buf, acc_sh):
    _sc_partial_kernel(src_hbm, idx_hbm, part_hbm, src_buf, idx_buf, acc_sh)


def _combine_body(p_ref, o_ref, out_ref):
    out_ref[...] = p_ref[0] + p_ref[1] + o_ref[...]


_ROWS_BLK = 400


def _combine(part, out):
    return pl.pallas_call(
        _combine_body,
        grid=(N_NODES // _ROWS_BLK,),
        in_specs=[
            pl.BlockSpec((NC, _ROWS_BLK, D), lambda i: (0, i, 0)),
            pl.BlockSpec((_ROWS_BLK, D), lambda i: (i, 0)),
        ],
        out_specs=pl.BlockSpec((_ROWS_BLK, D), lambda i: (i, 0)),
        out_shape=jax.ShapeDtypeStruct((N_NODES, D), jnp.float32),
    )(part, out)


@jax.jit
def kernel(src, index, out):
    idx = index.astype(jnp.int32)
    part = _sc_partial(src, idx)
    return _combine(part, out)


# speedup vs baseline: 7.2172x; 1.9636x over previous
import functools

import jax
import jax.numpy as jnp
from jax import lax
from jax.experimental import pallas as pl
from jax.experimental.pallas import tpu as pltpu
from jax.experimental.pallas import tpu_sc as plsc

N_EDGES = 320000
N_NODES = 10000
D = 128

NC = 2
NS = 16
N_PAD = 10240
RPC = N_PAD // NC
RPT = RPC // NS
GARBAGE = RPC
ACC_ROWS = RPC + 8
CHUNK = 256
NCH16 = N_EDGES // 16


def _sc_kernel(src_hbm, idx_hbm, out_in_hbm, out_hbm,
               src_buf, idx_buf, probe, acc_sh, sem):
    c = lax.axis_index("c")
    s = lax.axis_index("s")

    target = jnp.int32(RPC)

    def bs_body(i, st):
        lo, hi = st
        mid = (lo + hi) // 2
        pltpu.sync_copy(idx_hbm.at[pl.ds(mid * 16, 16)], probe)
        take = probe[...][0] < target
        return (jnp.where(take, mid, lo), jnp.where(take, hi, mid))

    lo16, _ = lax.fori_loop(0, 15, bs_body, (jnp.int32(0), jnp.int32(NCH16)))
    pltpu.sync_copy(idx_hbm.at[pl.ds(lo16 * 16, 16)], probe)
    pv = probe[...]
    below = jnp.int32(0)
    for k in range(16):
        below = below + jnp.where(pv[k] < target, 1, 0)
    split = lo16 * 16 + below

    g0 = c * RPC + s * RPT

    @pl.when(g0 + RPT <= N_NODES)
    def _():
        pltpu.sync_copy(out_in_hbm.at[pl.ds(g0, RPT)],
                        acc_sh.at[pl.ds(s * RPT, RPT)])

    @pl.when(jnp.logical_and(g0 < N_NODES, g0 + RPT > N_NODES))
    def _():
        pltpu.sync_copy(out_in_hbm.at[pl.ds(g0, N_NODES - (NC * NS - 1) * RPT)],
                        acc_sh.at[pl.ds(s * RPT, N_NODES - (NC * NS - 1) * RPT)])

    plsc.subcore_barrier()

    e_lo = jnp.where(c == 0, 0, split)
    e_hi = jnp.where(c == 0, split, N_EDGES)
    per = (e_hi - e_lo + NS - 1) // NS
    my_lo = e_lo + s * per
    my_hi = jnp.minimum(my_lo + per, e_hi)
    my_lo = jnp.minimum(my_lo, my_hi)
    a0 = (my_lo // 8) * 8
    n_chunks = (my_hi - a0 + CHUNK - 1) // CHUNK
    nbase = c * RPC

    def start_loads(t, b):
        e_c = jnp.minimum(a0 + t * CHUNK, N_EDGES - CHUNK)
        pltpu.async_copy(src_hbm.at[pl.ds(e_c, CHUNK), :], src_buf.at[b], sem)
        for h in range(CHUNK // 128):
            pltpu.async_copy(idx_hbm.at[pl.ds(e_c + h * 128, 128)],
                             idx_buf.at[b, h], sem)

    def wait_loads(t, b):
        e_c = jnp.minimum(a0 + t * CHUNK, N_EDGES - CHUNK)
        pltpu.make_async_copy(src_hbm.at[pl.ds(e_c, CHUNK), :],
                              src_buf.at[b], sem).wait()
        for h in range(CHUNK // 128):
            pltpu.make_async_copy(idx_hbm.at[pl.ds(e_c + h * 128, 128)],
                                  idx_buf.at[b, h], sem).wait()

    @pl.when(n_chunks > 0)
    def _():
        start_loads(0, 0)

    iota16 = lax.iota(jnp.int32, 16)

    def chunk_body(t, _):
        b = t % 2
        wait_loads(t, b)

        @pl.when(t + 1 < n_chunks)
        def _():
            start_loads(t + 1, 1 - b)

        e_c = jnp.minimum(a0 + t * CHUNK, N_EDGES - CHUNK)
        lmax = jnp.maximum(my_lo, a0 + t * CHUNK)
        for h in range(CHUNK // 128):
            for k in range(8):
                pos = e_c + h * 128 + k * 16 + iota16
                v = idx_buf[b, h, pl.ds(k * 16, 16)]
                ok = jnp.logical_and(pos >= lmax, pos < my_hi)
                idx_buf[b, h, pl.ds(k * 16, 16)] = jnp.where(
                    ok, v - nbase, jnp.int32(GARBAGE))
        for h in range(CHUNK // 128):
            pltpu.sync_copy(src_buf.at[b, pl.ds(h * 128, 128), :],
                            acc_sh.at[idx_buf.at[b, h]],
                            add=True)
        return 0

    lax.fori_loop(0, n_chunks, chunk_body, 0)

    plsc.subcore_barrier()

    pltpu.sync_copy(acc_sh.at[pl.ds(s * RPT, RPT)],
                    out_hbm.at[pl.ds(c * RPC + s * RPT, RPT)])


@functools.partial(
    pl.kernel,
    mesh=plsc.VectorSubcoreMesh(core_axis_name="c", subcore_axis_name="s"),
    out_type=jax.ShapeDtypeStruct((N_PAD, D), jnp.float32),
    scratch_types=[
        pltpu.VMEM((2, CHUNK, D), jnp.float32),
        pltpu.VMEM((2, CHUNK // 128, 128), jnp.int32),
        pltpu.VMEM((16,), jnp.int32),
        pltpu.VMEM_SHARED((ACC_ROWS, D), jnp.float32),
        pltpu.SemaphoreType.DMA,
    ],
)
def _sc_scatter(src_hbm, idx_hbm, out_in_hbm, out_hbm,
                src_buf, idx_buf, probe, acc_sh, sem):
    _sc_kernel(src_hbm, idx_hbm, out_in_hbm, out_hbm,
               src_buf, idx_buf, probe, acc_sh, sem)


@jax.jit
def kernel(src, index, out):
    idx = index.astype(jnp.int32)
    res = _sc_scatter(src, idx, out)
    return res[:N_NODES]


# speedup vs baseline: 7.2991x; 1.0113x over previous
import functools

import jax
import jax.numpy as jnp
from jax import lax
from jax.experimental import pallas as pl
from jax.experimental.pallas import tpu as pltpu
from jax.experimental.pallas import tpu_sc as plsc

N_EDGES = 320000
N_NODES = 10000
D = 128

NC = 2
NS = 16
N_PAD = 10240
RPC = N_PAD // NC
RPT = RPC // NS
GARBAGE = RPC
ACC_ROWS = RPC + 8
CHUNK = 256
NCH16 = N_EDGES // 16


def _sc_kernel(src_hbm, idx_hbm, out_in_hbm, out_hbm,
               src_buf, idx_buf, probe, acc_sh, sem):
    c = lax.axis_index("c")
    s = lax.axis_index("s")

    g0 = c * RPC + s * RPT
    n_last = N_NODES - (NC * NS - 1) * RPT

    @pl.when(g0 + RPT <= N_NODES)
    def _():
        pltpu.async_copy(out_in_hbm.at[pl.ds(g0, RPT)],
                         acc_sh.at[pl.ds(s * RPT, RPT)], sem)

    @pl.when(jnp.logical_and(g0 < N_NODES, g0 + RPT > N_NODES))
    def _():
        pltpu.async_copy(out_in_hbm.at[pl.ds(g0, n_last)],
                         acc_sh.at[pl.ds(s * RPT, n_last)], sem)

    target = jnp.int32(RPC)

    def bs_body(i, st):
        lo, hi = st
        mid = (lo + hi) // 2
        pltpu.sync_copy(idx_hbm.at[pl.ds(mid * 16, 16)], probe)
        take = probe[...][0] < target
        return (jnp.where(take, mid, lo), jnp.where(take, hi, mid))

    lo16, _ = lax.fori_loop(0, 15, bs_body, (jnp.int32(0), jnp.int32(NCH16)))
    pltpu.sync_copy(idx_hbm.at[pl.ds(lo16 * 16, 16)], probe)
    pv = probe[...]
    below = jnp.int32(0)
    for k in range(16):
        below = below + jnp.where(pv[k] < target, 1, 0)
    split = lo16 * 16 + below

    @pl.when(g0 + RPT <= N_NODES)
    def _():
        pltpu.make_async_copy(out_in_hbm.at[pl.ds(g0, RPT)],
                              acc_sh.at[pl.ds(s * RPT, RPT)], sem).wait()

    @pl.when(jnp.logical_and(g0 < N_NODES, g0 + RPT > N_NODES))
    def _():
        pltpu.make_async_copy(out_in_hbm.at[pl.ds(g0, n_last)],
                              acc_sh.at[pl.ds(s * RPT, n_last)], sem).wait()

    plsc.subcore_barrier()

    e_lo = jnp.where(c == 0, 0, split)
    e_hi = jnp.where(c == 0, split, N_EDGES)
    per = (e_hi - e_lo + NS - 1) // NS
    my_lo = e_lo + s * per
    my_hi = jnp.minimum(my_lo + per, e_hi)
    my_lo = jnp.minimum(my_lo, my_hi)
    a0 = (my_lo // 8) * 8
    n_chunks = (my_hi - a0 + CHUNK - 1) // CHUNK
    nbase = c * RPC

    def start_loads(t, b):
        e_c = jnp.minimum(a0 + t * CHUNK, N_EDGES - CHUNK)
        pltpu.async_copy(src_hbm.at[pl.ds(e_c, CHUNK), :], src_buf.at[b], sem)
        for h in range(CHUNK // 128):
            pltpu.async_copy(idx_hbm.at[pl.ds(e_c + h * 128, 128)],
                             idx_buf.at[b, h], sem)

    def wait_loads(t, b):
        e_c = jnp.minimum(a0 + t * CHUNK, N_EDGES - CHUNK)
        pltpu.make_async_copy(src_hbm.at[pl.ds(e_c, CHUNK), :],
                              src_buf.at[b], sem).wait()
        for h in range(CHUNK // 128):
            pltpu.make_async_copy(idx_hbm.at[pl.ds(e_c + h * 128, 128)],
                                  idx_buf.at[b, h], sem).wait()

    @pl.when(n_chunks > 0)
    def _():
        start_loads(0, 0)

    iota16 = lax.iota(jnp.int32, 16)

    def chunk_body(t, _):
        b = t % 2
        wait_loads(t, b)

        @pl.when(t + 1 < n_chunks)
        def _():
            start_loads(t + 1, 1 - b)

        e_c = jnp.minimum(a0 + t * CHUNK, N_EDGES - CHUNK)
        lmax = jnp.maximum(my_lo, a0 + t * CHUNK)
        for h in range(CHUNK // 128):
            for k in range(8):
                pos = e_c + h * 128 + k * 16 + iota16
                v = idx_buf[b, h, pl.ds(k * 16, 16)]
                ok = jnp.logical_and(pos >= lmax, pos < my_hi)
                idx_buf[b, h, pl.ds(k * 16, 16)] = jnp.where(
                    ok, v - nbase, jnp.int32(GARBAGE))
        for h in range(CHUNK // 128):
            pltpu.sync_copy(src_buf.at[b, pl.ds(h * 128, 128), :],
                            acc_sh.at[idx_buf.at[b, h]],
                            add=True)
        return 0

    lax.fori_loop(0, n_chunks, chunk_body, 0)

    plsc.subcore_barrier()

    pltpu.sync_copy(acc_sh.at[pl.ds(s * RPT, RPT)],
                    out_hbm.at[pl.ds(c * RPC + s * RPT, RPT)])


@functools.partial(
    pl.kernel,
    mesh=plsc.VectorSubcoreMesh(core_axis_name="c", subcore_axis_name="s"),
    out_type=jax.ShapeDtypeStruct((N_PAD, D), jnp.float32),
    scratch_types=[
        pltpu.VMEM((2, CHUNK, D), jnp.float32),
        pltpu.VMEM((2, CHUNK // 128, 128), jnp.int32),
        pltpu.VMEM((16,), jnp.int32),
        pltpu.VMEM_SHARED((ACC_ROWS, D), jnp.float32),
        pltpu.SemaphoreType.DMA,
    ],
)
def _sc_scatter(src_hbm, idx_hbm, out_in_hbm, out_hbm,
                src_buf, idx_buf, probe, acc_sh, sem):
    _sc_kernel(src_hbm, idx_hbm, out_in_hbm, out_hbm,
               src_buf, idx_buf, probe, acc_sh, sem)


@jax.jit
def kernel(src, index, out):
    idx = index.astype(jnp.int32)
    res = _sc_scatter(src, idx, out)
    return res[:N_NODES]


# speedup vs baseline: 7.4189x; 1.0164x over previous
import functools

import jax
import jax.numpy as jnp
from jax import lax
from jax.experimental import pallas as pl
from jax.experimental.pallas import tpu as pltpu
from jax.experimental.pallas import tpu_sc as plsc

N_EDGES = 320000
N_NODES = 10000
D = 128

NC = 2
NS = 16
N_PAD = 10240
RPC = N_PAD // NC
RPT = RPC // NS
GARBAGE = RPC
ACC_ROWS = RPC + 8
CHUNK = 256
NCH16 = N_EDGES // 16


def _sc_kernel(src_hbm, idx_hbm, out_in_hbm, out_hbm,
               src_buf, idx_buf, probe, acc_sh, sem, sem_sc):
    c = lax.axis_index("c")
    s = lax.axis_index("s")

    g0 = c * RPC + s * RPT
    n_last = N_NODES - (NC * NS - 1) * RPT

    @pl.when(g0 + RPT <= N_NODES)
    def _():
        pltpu.async_copy(out_in_hbm.at[pl.ds(g0, RPT)],
                         acc_sh.at[pl.ds(s * RPT, RPT)], sem)

    @pl.when(jnp.logical_and(g0 < N_NODES, g0 + RPT > N_NODES))
    def _():
        pltpu.async_copy(out_in_hbm.at[pl.ds(g0, n_last)],
                         acc_sh.at[pl.ds(s * RPT, n_last)], sem)

    target = jnp.int32(RPC)

    def bs_body(i, st):
        lo, hi = st
        mid = (lo + hi) // 2
        pltpu.sync_copy(idx_hbm.at[pl.ds(mid * 16, 16)], probe)
        take = probe[...][0] < target
        return (jnp.where(take, mid, lo), jnp.where(take, hi, mid))

    lo16, _ = lax.fori_loop(0, 15, bs_body, (jnp.int32(0), jnp.int32(NCH16)))
    pltpu.sync_copy(idx_hbm.at[pl.ds(lo16 * 16, 16)], probe)
    pv = probe[...]
    below = jnp.int32(0)
    for k in range(16):
        below = below + jnp.where(pv[k] < target, 1, 0)
    split = lo16 * 16 + below

    @pl.when(g0 + RPT <= N_NODES)
    def _():
        pltpu.make_async_copy(out_in_hbm.at[pl.ds(g0, RPT)],
                              acc_sh.at[pl.ds(s * RPT, RPT)], sem).wait()

    @pl.when(jnp.logical_and(g0 < N_NODES, g0 + RPT > N_NODES))
    def _():
        pltpu.make_async_copy(out_in_hbm.at[pl.ds(g0, n_last)],
                              acc_sh.at[pl.ds(s * RPT, n_last)], sem).wait()

    plsc.subcore_barrier()

    e_lo = jnp.where(c == 0, 0, split)
    e_hi = jnp.where(c == 0, split, N_EDGES)
    per = (e_hi - e_lo + NS - 1) // NS
    my_lo = e_lo + s * per
    my_hi = jnp.minimum(my_lo + per, e_hi)
    my_lo = jnp.minimum(my_lo, my_hi)
    a0 = (my_lo // 8) * 8
    n_chunks = (my_hi - a0 + CHUNK - 1) // CHUNK
    nbase = c * RPC

    def start_loads(t, b):
        e_c = jnp.minimum(a0 + t * CHUNK, N_EDGES - CHUNK)
        pltpu.async_copy(src_hbm.at[pl.ds(e_c, CHUNK), :], src_buf.at[b], sem)
        for h in range(CHUNK // 128):
            pltpu.async_copy(idx_hbm.at[pl.ds(e_c + h * 128, 128)],
                             idx_buf.at[b, h], sem)

    def wait_loads(t, b):
        e_c = jnp.minimum(a0 + t * CHUNK, N_EDGES - CHUNK)
        pltpu.make_async_copy(src_hbm.at[pl.ds(e_c, CHUNK), :],
                              src_buf.at[b], sem).wait()
        for h in range(CHUNK // 128):
            pltpu.make_async_copy(idx_hbm.at[pl.ds(e_c + h * 128, 128)],
                                  idx_buf.at[b, h], sem).wait()

    @pl.when(n_chunks > 0)
    def _():
        start_loads(0, 0)

    iota16 = lax.iota(jnp.int32, 16)

    def start_scatter(b):
        for h in range(CHUNK // 128):
            pltpu.async_copy(src_buf.at[b, pl.ds(h * 128, 128), :],
                             acc_sh.at[idx_buf.at[b, h]], sem_sc, add=True)

    def wait_scatter(b):
        for h in range(CHUNK // 128):
            pltpu.make_async_copy(src_buf.at[b, pl.ds(h * 128, 128), :],
                                  acc_sh.at[idx_buf.at[b, h]], sem_sc).wait()

    def chunk_body(t, _):
        b = t % 2
        wait_loads(t, b)

        e_c = jnp.minimum(a0 + t * CHUNK, N_EDGES - CHUNK)
        lmax = jnp.maximum(my_lo, a0 + t * CHUNK)
        for h in range(CHUNK // 128):
            for k in range(8):
                pos = e_c + h * 128 + k * 16 + iota16
                v = idx_buf[b, h, pl.ds(k * 16, 16)]
                ok = jnp.logical_and(pos >= lmax, pos < my_hi)
                idx_buf[b, h, pl.ds(k * 16, 16)] = jnp.where(
                    ok, v - nbase, jnp.int32(GARBAGE))

        @pl.when(t >= 1)
        def _():
            wait_scatter(1 - b)

        @pl.when(t + 1 < n_chunks)
        def _():
            start_loads(t + 1, 1 - b)

        start_scatter(b)
        return 0

    lax.fori_loop(0, n_chunks, chunk_body, 0)

    @pl.when(n_chunks > 0)
    def _():
        wait_scatter((n_chunks - 1) % 2)

    plsc.subcore_barrier()

    pltpu.sync_copy(acc_sh.at[pl.ds(s * RPT, RPT)],
                    out_hbm.at[pl.ds(c * RPC + s * RPT, RPT)])


@functools.partial(
    pl.kernel,
    mesh=plsc.VectorSubcoreMesh(core_axis_name="c", subcore_axis_name="s"),
    out_type=jax.ShapeDtypeStruct((N_PAD, D), jnp.float32),
    scratch_types=[
        pltpu.VMEM((2, CHUNK, D), jnp.float32),
        pltpu.VMEM((2, CHUNK // 128, 128), jnp.int32),
        pltpu.VMEM((16,), jnp.int32),
        pltpu.VMEM_SHARED((ACC_ROWS, D), jnp.float32),
        pltpu.SemaphoreType.DMA,
        pltpu.SemaphoreType.DMA,
    ],
)
def _sc_scatter(src_hbm, idx_hbm, out_in_hbm, out_hbm,
                src_buf, idx_buf, probe, acc_sh, sem, sem_sc):
    _sc_kernel(src_hbm, idx_hbm, out_in_hbm, out_hbm,
               src_buf, idx_buf, probe, acc_sh, sem, sem_sc)


@jax.jit
def kernel(src, index, out):
    idx = index.astype(jnp.int32)
    res = _sc_scatter(src, idx, out)
    return res[:N_NODES]


# speedup vs baseline: 7.6989x; 1.0377x over previous
import functools

import jax
import jax.numpy as jnp
from jax import lax
from jax.experimental import pallas as pl
from jax.experimental.pallas import tpu as pltpu
from jax.experimental.pallas import tpu_sc as plsc

N_EDGES = 320000
N_NODES = 10000
D = 128

NC = 2
NS = 16
N_PAD = 10240
RPC = N_PAD // NC
RPT = RPC // NS
GARBAGE = RPC
ACC_ROWS = RPC + 8
CHUNK = 256
NCH16 = N_EDGES // 16


def _sc_kernel(src_hbm, idx_hbm, out_in_hbm, out_hbm,
               src_buf, idx_buf, probe, acc_sh, sem, sem_sc):
    c = lax.axis_index("c")
    s = lax.axis_index("s")

    g0 = c * RPC + s * RPT
    n_last = N_NODES - (NC * NS - 1) * RPT

    @pl.when(g0 + RPT <= N_NODES)
    def _():
        pltpu.async_copy(out_in_hbm.at[pl.ds(g0, RPT)],
                         acc_sh.at[pl.ds(s * RPT, RPT)], sem)

    @pl.when(jnp.logical_and(g0 < N_NODES, g0 + RPT > N_NODES))
    def _():
        pltpu.async_copy(out_in_hbm.at[pl.ds(g0, n_last)],
                         acc_sh.at[pl.ds(s * RPT, n_last)], sem)

    target = jnp.int32(RPC)

    def bs_body(i, st):
        lo, hi = st
        mid = (lo + hi) // 2
        pltpu.sync_copy(idx_hbm.at[pl.ds(mid * 16, 16)], probe)
        take = probe[...][0] < target
        return (jnp.where(take, mid, lo), jnp.where(take, hi, mid))

    lo16, _ = lax.fori_loop(0, 15, bs_body, (jnp.int32(0), jnp.int32(NCH16)))
    pltpu.sync_copy(idx_hbm.at[pl.ds(lo16 * 16, 16)], probe)
    pv = probe[...]
    below = jnp.int32(0)
    for k in range(16):
        below = below + jnp.where(pv[k] < target, 1, 0)
    split = lo16 * 16 + below

    @pl.when(g0 + RPT <= N_NODES)
    def _():
        pltpu.make_async_copy(out_in_hbm.at[pl.ds(g0, RPT)],
                              acc_sh.at[pl.ds(s * RPT, RPT)], sem).wait()

    @pl.when(jnp.logical_and(g0 < N_NODES, g0 + RPT > N_NODES))
    def _():
        pltpu.make_async_copy(out_in_hbm.at[pl.ds(g0, n_last)],
                              acc_sh.at[pl.ds(s * RPT, n_last)], sem).wait()

    plsc.subcore_barrier()

    e_lo = jnp.where(c == 0, 0, split)
    e_hi = jnp.where(c == 0, split, N_EDGES)
    per = (e_hi - e_lo + NS - 1) // NS
    my_lo = e_lo + s * per
    my_hi = jnp.minimum(my_lo + per, e_hi)
    my_lo = jnp.minimum(my_lo, my_hi)
    a0 = (my_lo // 8) * 8
    n_chunks = (my_hi - a0 + CHUNK - 1) // CHUNK
    nbase = c * RPC

    def start_loads(t, b):
        e_c = jnp.minimum(a0 + t * CHUNK, N_EDGES - CHUNK)
        pltpu.async_copy(src_hbm.at[pl.ds(e_c, CHUNK), :], src_buf.at[b], sem)
        for h in range(CHUNK // 128):
            pltpu.async_copy(idx_hbm.at[pl.ds(e_c + h * 128, 128)],
                             idx_buf.at[b, h], sem)

    def wait_loads(t, b):
        e_c = jnp.minimum(a0 + t * CHUNK, N_EDGES - CHUNK)
        pltpu.make_async_copy(src_hbm.at[pl.ds(e_c, CHUNK), :],
                              src_buf.at[b], sem).wait()
        for h in range(CHUNK // 128):
            pltpu.make_async_copy(idx_hbm.at[pl.ds(e_c + h * 128, 128)],
                                  idx_buf.at[b, h], sem).wait()

    @pl.when(n_chunks > 0)
    def _():
        start_loads(0, 0)

    iota16 = lax.iota(jnp.int32, 16)

    def start_scatter(b):
        for h in range(CHUNK // 128):
            pltpu.async_copy(src_buf.at[b, pl.ds(h * 128, 128), :],
                             acc_sh.at[idx_buf.at[b, h]], sem_sc, add=True)

    def wait_scatter(b):
        for h in range(CHUNK // 128):
            pltpu.make_async_copy(src_buf.at[b, pl.ds(h * 128, 128), :],
                                  acc_sh.at[idx_buf.at[b, h]], sem_sc).wait()

    def chunk_body(t, _):
        b = t % 2
        wait_loads(t, b)

        e_c = jnp.minimum(a0 + t * CHUNK, N_EDGES - CHUNK)
        lmax = jnp.maximum(my_lo, a0 + t * CHUNK)
        for h in range(CHUNK // 128):
            for k in range(8):
                pos = e_c + h * 128 + k * 16 + iota16
                v = idx_buf[b, h, pl.ds(k * 16, 16)]
                ok = jnp.logical_and(pos >= lmax, pos < my_hi)
                idx_buf[b, h, pl.ds(k * 16, 16)] = jnp.where(
                    ok, v - nbase, jnp.int32(GARBAGE))

        @pl.when(t >= 1)
        def _():
            wait_scatter(1 - b)

        @pl.when(t + 1 < n_chunks)
        def _():
            start_loads(t + 1, 1 - b)

        start_scatter(b)
        return 0

    lax.fori_loop(0, n_chunks, chunk_body, 0)

    @pl.when(n_chunks > 0)
    def _():
        wait_scatter((n_chunks - 1) % 2)

    plsc.subcore_barrier()

    @pl.when(g0 + RPT <= N_NODES)
    def _():
        pltpu.sync_copy(acc_sh.at[pl.ds(s * RPT, RPT)],
                        out_hbm.at[pl.ds(g0, RPT)])

    @pl.when(jnp.logical_and(g0 < N_NODES, g0 + RPT > N_NODES))
    def _():
        pltpu.sync_copy(acc_sh.at[pl.ds(s * RPT, n_last)],
                        out_hbm.at[pl.ds(g0, n_last)])


@functools.partial(
    pl.kernel,
    mesh=plsc.VectorSubcoreMesh(core_axis_name="c", subcore_axis_name="s"),
    out_type=jax.ShapeDtypeStruct((N_NODES, D), jnp.float32),
    scratch_types=[
        pltpu.VMEM((2, CHUNK, D), jnp.float32),
        pltpu.VMEM((2, CHUNK // 128, 128), jnp.int32),
        pltpu.VMEM((16,), jnp.int32),
        pltpu.VMEM_SHARED((ACC_ROWS, D), jnp.float32),
        pltpu.SemaphoreType.DMA,
        pltpu.SemaphoreType.DMA,
    ],
)
def _sc_scatter(src_hbm, idx_hbm, out_in_hbm, out_hbm,
                src_buf, idx_buf, probe, acc_sh, sem, sem_sc):
    _sc_kernel(src_hbm, idx_hbm, out_in_hbm, out_hbm,
               src_buf, idx_buf, probe, acc_sh, sem, sem_sc)


@jax.jit
def kernel(src, index, out):
    idx = index.astype(jnp.int32)
    return _sc_scatter(src, idx, out)


# speedup vs baseline: 7.9547x; 1.0332x over previous
import functools

import jax
import jax.numpy as jnp
from jax import lax
from jax.experimental import pallas as pl
from jax.experimental.pallas import tpu as pltpu
from jax.experimental.pallas import tpu_sc as plsc

N_EDGES = 320000
N_NODES = 10000
D = 128

NC = 2
NS = 16
NT = NC * NS
RPT = 320
RPC = NS * RPT
GARBAGE = RPC
ACC_ROWS = RPC + 8
CHUNK = 256
NCH16 = N_EDGES // 16


def _sc_kernel(src_hbm, idx_hbm, out_in_hbm, out_hbm,
               src_buf, idx_buf, probe_a, probe_b, acc_sh,
               sem, sem_sc, sem_init, sem_probe):
    c = lax.axis_index("c")
    s = lax.axis_index("s")
    n0 = (c * NS + s) * RPT
    r0 = s * RPT
    n_rows = jnp.minimum(jnp.int32(N_NODES) - n0, RPT)
    nbase = c * RPC
    g_row = jnp.int32(GARBAGE) + (s % 8)

    pltpu.async_copy(out_in_hbm.at[pl.ds(n0, n_rows)],
                     acc_sh.at[pl.ds(r0, n_rows)], sem_init)

    ta = n0
    tb = n0 + RPT

    def bs_body(i, st):
        lo_a, hi_a, lo_b, hi_b = st
        mid_a = (lo_a + hi_a) // 2
        mid_b = (lo_b + hi_b) // 2
        pltpu.async_copy(idx_hbm.at[pl.ds(mid_a * 16, 16)], probe_a, sem_probe)
        pltpu.async_copy(idx_hbm.at[pl.ds(mid_b * 16, 16)], probe_b, sem_probe)
        pltpu.make_async_copy(idx_hbm.at[pl.ds(mid_a * 16, 16)], probe_a,
                              sem_probe).wait()
        pltpu.make_async_copy(idx_hbm.at[pl.ds(mid_b * 16, 16)], probe_b,
                              sem_probe).wait()
        take_a = probe_a[...][0] < ta
        take_b = probe_b[...][0] < tb
        return (jnp.where(take_a, mid_a, lo_a), jnp.where(take_a, hi_a, mid_a),
                jnp.where(take_b, mid_b, lo_b), jnp.where(take_b, hi_b, mid_b))

    lo_a, _, lo_b, _ = lax.fori_loop(
        0, 15, bs_body,
        (jnp.int32(0), jnp.int32(NCH16), jnp.int32(0), jnp.int32(NCH16)))
    pltpu.async_copy(idx_hbm.at[pl.ds(lo_a * 16, 16)], probe_a, sem_probe)
    pltpu.async_copy(idx_hbm.at[pl.ds(lo_b * 16, 16)], probe_b, sem_probe)
    pltpu.make_async_copy(idx_hbm.at[pl.ds(lo_a * 16, 16)], probe_a,
                          sem_probe).wait()
    pltpu.make_async_copy(idx_hbm.at[pl.ds(lo_b * 16, 16)], probe_b,
                          sem_probe).wait()
    pa = probe_a[...]
    pb = probe_b[...]
    below_a = jnp.int32(0)
    below_b = jnp.int32(0)
    for k in range(16):
        below_a = below_a + jnp.where(pa[k] < ta, 1, 0)
        below_b = below_b + jnp.where(pb[k] < tb, 1, 0)
    my_lo = lo_a * 16 + below_a
    my_hi = lo_b * 16 + below_b

    pltpu.make_async_copy(out_in_hbm.at[pl.ds(n0, n_rows)],
                          acc_sh.at[pl.ds(r0, n_rows)], sem_init).wait()

    a0 = (my_lo // 8) * 8
    n_chunks = (my_hi - a0 + CHUNK - 1) // CHUNK

    def start_loads(t, b):
        e_c = jnp.minimum(a0 + t * CHUNK, N_EDGES - CHUNK)
        pltpu.async_copy(src_hbm.at[pl.ds(e_c, CHUNK), :], src_buf.at[b], sem)
        for h in range(CHUNK // 128):
            pltpu.async_copy(idx_hbm.at[pl.ds(e_c + h * 128, 128)],
                             idx_buf.at[b, h], sem)

    def wait_loads(t, b):
        e_c = jnp.minimum(a0 + t * CHUNK, N_EDGES - CHUNK)
        pltpu.make_async_copy(src_hbm.at[pl.ds(e_c, CHUNK), :],
                              src_buf.at[b], sem).wait()
        for h in range(CHUNK // 128):
            pltpu.make_async_copy(idx_hbm.at[pl.ds(e_c + h * 128, 128)],
                                  idx_buf.at[b, h], sem).wait()

    @pl.when(n_chunks > 0)
    def _():
        start_loads(0, 0)

    iota16 = lax.iota(jnp.int32, 16)

    def start_scatter(b):
        for h in range(CHUNK // 128):
            pltpu.async_copy(src_buf.at[b, pl.ds(h * 128, 128), :],
                             acc_sh.at[idx_buf.at[b, h]], sem_sc, add=True)

    def wait_scatter(b):
        for h in range(CHUNK // 128):
            pltpu.make_async_copy(src_buf.at[b, pl.ds(h * 128, 128), :],
                                  acc_sh.at[idx_buf.at[b, h]], sem_sc).wait()

    def chunk_body(t, _):
        b = t % 2
        wait_loads(t, b)

        e_c = jnp.minimum(a0 + t * CHUNK, N_EDGES - CHUNK)
        lmax = jnp.maximum(my_lo, a0 + t * CHUNK)
        for h in range(CHUNK // 128):
            for k in range(8):
                pos = e_c + h * 128 + k * 16 + iota16
                v = idx_buf[b, h, pl.ds(k * 16, 16)]
                ok = jnp.logical_and(pos >= lmax, pos < my_hi)
                idx_buf[b, h, pl.ds(k * 16, 16)] = jnp.where(
                    ok, v - nbase, g_row)

        @pl.when(t >= 1)
        def _():
            wait_scatter(1 - b)

        @pl.when(t + 1 < n_chunks)
        def _():
            start_loads(t + 1, 1 - b)

        start_scatter(b)
        return 0

    lax.fori_loop(0, n_chunks, chunk_body, 0)

    @pl.when(n_chunks > 0)
    def _():
        wait_scatter((n_chunks - 1) % 2)

    pltpu.sync_copy(acc_sh.at[pl.ds(r0, n_rows)], out_hbm.at[pl.ds(n0, n_rows)])


@functools.partial(
    pl.kernel,
    mesh=plsc.VectorSubcoreMesh(core_axis_name="c", subcore_axis_name="s"),
    out_type=jax.ShapeDtypeStruct((N_NODES, D), jnp.float32),
    scratch_types=[
        pltpu.VMEM((2, CHUNK, D), jnp.float32),
        pltpu.VMEM((2, CHUNK // 128, 128), jnp.int32),
        pltpu.VMEM((16,), jnp.int32),
        pltpu.VMEM((16,), jnp.int32),
        pltpu.VMEM_SHARED((ACC_ROWS, D), jnp.float32),
        pltpu.SemaphoreType.DMA,
        pltpu.SemaphoreType.DMA,
        pltpu.SemaphoreType.DMA,
        pltpu.SemaphoreType.DMA,
    ],
)
def _sc_scatter(src_hbm, idx_hbm, out_in_hbm, out_hbm,
                src_buf, idx_buf, probe_a, probe_b, acc_sh,
                sem, sem_sc, sem_init, sem_probe):
    _sc_kernel(src_hbm, idx_hbm, out_in_hbm, out_hbm,
               src_buf, idx_buf, probe_a, probe_b, acc_sh,
               sem, sem_sc, sem_init, sem_probe)


@jax.jit
def kernel(src, index, out):
    idx = index.astype(jnp.int32)
    return _sc_scatter(src, idx, out)
